# baseline (device time: 30692 ns/iter reference)
import jax
import jax.numpy as jnp
from jax import lax
from jax.experimental import pallas as pl
from jax.experimental.pallas import tpu as pltpu

N_DEV = 4
N_TOK = 1024
D_MODEL = 256
D_FF = 512
N_EXP = 16
EXP_PER_DEV = N_EXP // N_DEV
HALF_ROW = N_TOK // 2
QUAR_ROW = N_TOK // 4
N_CH = 4
CW = D_FF // N_CH
BF = jnp.bfloat16
F32 = jnp.float32


def kernel(x, router_W, route_idx, expert_W):
    def body(x_ref, rw_ref, idx_ref, ew_ref, out_ref,
             comm1_ref, comm2_ref, sbuf1_ref, sbuf2_ref, hbuf_ref, obuf_ref,
             send_sems, recv_sems):
        my = lax.axis_index("i")
        x_c = (my >= 2).astype(jnp.int32)
        y_c = ((my == 1) | (my == 2)).astype(jnp.int32)
        p_y = my ^ 1
        p_x = 3 - my

        barrier_sem = pltpu.get_barrier_semaphore()
        for nbr in (p_y, p_x):
            pl.semaphore_signal(
                barrier_sem, inc=1,
                device_id=(nbr,), device_id_type=pl.DeviceIdType.MESH,
            )
        pl.semaphore_wait(barrier_sem, 2)

        xv = x_ref[:, :]
        scores = jnp.dot(xv, rw_ref[:, :], preferred_element_type=F32)
        m = jnp.max(scores, axis=-1, keepdims=True)
        p = jnp.exp(scores - m)
        p = p / jnp.sum(p, axis=-1, keepdims=True)
        e0 = idx_ref[:, 0:1]
        e1 = idx_ref[:, 1:2]
        eids = lax.broadcasted_iota(jnp.int32, (N_TOK, N_EXP), 1)
        g0 = jnp.sum(jnp.where(eids == e0, p, 0.0), axis=-1, keepdims=True)
        g1 = jnp.sum(jnp.where(eids == e1, p, 0.0), axis=-1, keepdims=True)
        gs = g0 + g1

        xw = []
        for j in range(EXP_PER_DEV):
            e = my * EXP_PER_DEV + j
            w = (jnp.where(e0 == e, g0, 0.0) + jnp.where(e1 == e, g1, 0.0)) / gs
            xw.append((xv * w).astype(BF))

        G = []
        for ch in range(N_CH):
            if ch < 2:
                v1, v2, prt = y_c, x_c, [p_y, p_x, p_x, p_y]
            else:
                v1, v2, prt = x_c, y_c, [p_x, p_y, p_y, p_x]
            keep1 = v1 * HALF_ROW
            G.append(dict(
                v1=v1, v2=v2, prt=prt, c0=ch * CW,
                keep1=keep1,
                send1=(1 - v1) * HALF_ROW,
                keep2=keep1 + v2 * QUAR_ROW,
                send2=keep1 + (1 - v2) * QUAR_ROW,
                off2=v2 * QUAR_ROW,
                roff2=(1 - v2) * QUAR_ROW,
            ))

        def make_rdma(ch, stage, src, dst):
            k = ch * 4 + stage
            return pltpu.make_async_remote_copy(
                src_ref=src,
                dst_ref=dst,
                send_sem=send_sems.at[k],
                recv_sem=recv_sems.at[k],
                device_id=(G[ch]["prt"][stage],),
                device_id_type=pl.DeviceIdType.MESH,
            )

        def cols(ch):
            return pl.ds(G[ch]["c0"], CW)

        def rs1_start(ch):
            g = G[ch]
            sbuf1_ref[ch, :, :] = out_ref[pl.ds(g["send1"], HALF_ROW), cols(ch)].astype(BF)
            r = make_rdma(ch, 0, sbuf1_ref.at[ch], comm1_ref.at[ch])
            r.start()
            return r

        def rs1_fin(ch, r):
            g = G[ch]
            r.wait()
            sl = (pl.ds(g["keep1"], HALF_ROW), cols(ch))
            out_ref[sl] = out_ref[sl] + comm1_ref[ch].astype(F32)

        def rs2_start(ch):
            g = G[ch]
            sbuf2_ref[ch, :, :] = out_ref[pl.ds(g["send2"], QUAR_ROW), cols(ch)].astype(BF)
            r = make_rdma(ch, 1, sbuf2_ref.at[ch], comm2_ref.at[ch])
            r.start()
            return r

        def rs2_fin(ch, r):
            g = G[ch]
            r.wait()
            sl = (pl.ds(g["keep2"], QUAR_ROW), cols(ch))
            q = out_ref[sl] + comm2_ref[ch].astype(F32)
            out_ref[sl] = q
            hbuf_ref[ch, pl.ds(g["off2"], QUAR_ROW), :] = q.astype(BF)

        def ag1_start(ch):
            g = G[ch]
            sl = hbuf_ref.at[ch, pl.ds(g["off2"], QUAR_ROW), :]
            r = make_rdma(ch, 2, sl, sl)
            r.start()
            return r

        def ag1_fin(ch, r):
            g = G[ch]
            r.wait()
            out_ref[pl.ds(g["keep1"] + g["roff2"], QUAR_ROW), cols(ch)] = (
                hbuf_ref[ch, pl.ds(g["roff2"], QUAR_ROW), :].astype(F32)
            )

        def ag2_start(ch):
            r = make_rdma(ch, 3, hbuf_ref.at[ch], obuf_ref.at[ch])
            r.start()
            return r

        def ag2_fin(ch, r):
            g = G[ch]
            r.wait()
            out_ref[pl.ds(g["send1"], HALF_ROW), cols(ch)] = (
                obuf_ref[ch].astype(F32)
            )

        rd = [None] * N_CH
        for ch in (0, 2, 1, 3):
            c0 = G[ch]["c0"]
            partial = jnp.zeros((N_TOK, CW), F32)
            for j in range(EXP_PER_DEV):
                partial = partial + jnp.dot(
                    xw[j], ew_ref[j, :, c0:c0 + CW].astype(BF),
                    preferred_element_type=F32,
                )
            out_ref[:, c0:c0 + CW] = partial
            rd[ch] = rs1_start(ch)

        for ch in (0, 2, 1, 3):
            rs1_fin(ch, rd[ch])
            rd[ch] = rs2_start(ch)
        for ch in (0, 2, 1, 3):
            rs2_fin(ch, rd[ch])
            rd[ch] = ag1_start(ch)
        for ch in (0, 2, 1, 3):
            ag1_fin(ch, rd[ch])
            rd[ch] = ag2_start(ch)
        for ch in (0, 2, 1, 3):
            ag2_fin(ch, rd[ch])

    return pl.pallas_call(
        body,
        out_shape=jax.ShapeDtypeStruct((N_TOK, D_FF), F32),
        in_specs=[
            pl.BlockSpec(memory_space=pltpu.VMEM),
            pl.BlockSpec(memory_space=pltpu.VMEM),
            pl.BlockSpec(memory_space=pltpu.VMEM),
            pl.BlockSpec(memory_space=pltpu.VMEM),
        ],
        out_specs=pl.BlockSpec(memory_space=pltpu.VMEM),
        scratch_shapes=[
            pltpu.VMEM((N_CH, HALF_ROW, CW), BF),
            pltpu.VMEM((N_CH, QUAR_ROW, CW), BF),
            pltpu.VMEM((N_CH, HALF_ROW, CW), BF),
            pltpu.VMEM((N_CH, QUAR_ROW, CW), BF),
            pltpu.VMEM((N_CH, HALF_ROW, CW), BF),
            pltpu.VMEM((N_CH, HALF_ROW, CW), BF),
            pltpu.SemaphoreType.DMA((4 * N_CH,)),
            pltpu.SemaphoreType.DMA((4 * N_CH,)),
        ],
        compiler_params=pltpu.CompilerParams(collective_id=0),
    )(x, router_W, route_idx, expert_W)


# device time: 23964 ns/iter; 1.2808x vs baseline; 1.2808x over previous
import jax
import jax.numpy as jnp
from jax import lax
from jax.experimental import pallas as pl
from jax.experimental.pallas import tpu as pltpu

N_DEV = 4
N_TOK = 1024
D_MODEL = 256
D_FF = 512
N_EXP = 16
EXP_PER_DEV = N_EXP // N_DEV
HALF_ROW = N_TOK // 2
HALF_COL = D_FF // 2
N_PC = 4
PW = D_FF // N_PC
BF = jnp.bfloat16
F32 = jnp.float32


def kernel(x, router_W, route_idx, expert_W):
    def body(x_ref, rw_ref, idx_ref, ew_ref, out_ref,
             acc_ref, fbuf_ref, ewv_ref, comm1_ref, comm2_ref,
             send_sems, recv_sems, ew_sem, wb_sems):
        my = lax.axis_index("i")
        x_c = (my >= 2).astype(jnp.int32)
        y_c = ((my == 1) | (my == 2)).astype(jnp.int32)
        p_y = my ^ 1
        p_x = 3 - my

        ew_cp = pltpu.make_async_copy(ew_ref, ewv_ref, ew_sem)
        ew_cp.start()

        barrier_sem = pltpu.get_barrier_semaphore()
        for nbr in (p_y, p_x):
            pl.semaphore_signal(
                barrier_sem, inc=1,
                device_id=(nbr,), device_id_type=pl.DeviceIdType.MESH,
            )
        pl.semaphore_wait(barrier_sem, 2)

        xv = x_ref[:, :]
        scores = jnp.dot(xv, rw_ref[:, :], preferred_element_type=F32)
        m = jnp.max(scores, axis=-1, keepdims=True)
        p = jnp.exp(scores - m)
        p = p / jnp.sum(p, axis=-1, keepdims=True)
        e0 = idx_ref[:, 0:1]
        e1 = idx_ref[:, 1:2]
        eids = lax.broadcasted_iota(jnp.int32, (N_TOK, N_EXP), 1)
        g0 = jnp.sum(jnp.where(eids == e0, p, 0.0), axis=-1, keepdims=True)
        g1 = jnp.sum(jnp.where(eids == e1, p, 0.0), axis=-1, keepdims=True)
        gs = g0 + g1

        xw = []
        for j in range(EXP_PER_DEV):
            e = my * EXP_PER_DEV + j
            w = (jnp.where(e0 == e, g0, 0.0) + jnp.where(e1 == e, g1, 0.0)) / gs
            xw.append((xv * w).astype(BF))

        G = []
        for pc in range(N_PC):
            v1 = y_c if pc < 2 else x_c
            prt1 = p_y if pc < 2 else p_x
            prt2 = p_x if pc < 2 else p_y
            G.append(dict(
                prt=[prt1, prt2, prt1], c0=pc * PW,
                keep1=v1 * HALF_ROW,
                send1=(1 - v1) * HALF_ROW,
            ))

        def make_rdma(pc, stage, src, dst):
            k = pc * 3 + stage
            return pltpu.make_async_remote_copy(
                src_ref=src,
                dst_ref=dst,
                send_sem=send_sems.at[k],
                recv_sem=recv_sems.at[k],
                device_id=(G[pc]["prt"][stage],),
                device_id_type=pl.DeviceIdType.MESH,
            )

        def cols(pc):
            return pl.ds(G[pc]["c0"], PW)

        def keep_sl(pc):
            return acc_ref.at[pl.ds(G[pc]["keep1"], HALF_ROW), cols(pc)]

        def s1_start(pc):
            g = G[pc]
            r = make_rdma(
                pc, 0,
                acc_ref.at[pl.ds(g["send1"], HALF_ROW), cols(pc)],
                comm1_ref.at[pc],
            )
            r.start()
            return r

        def s1_fin(pc, r):
            g = G[pc]
            r.wait()
            sl = (pl.ds(g["keep1"], HALF_ROW), cols(pc))
            acc_ref[sl] = acc_ref[sl] + comm1_ref[pc]

        def s2_start(pc):
            r = make_rdma(pc, 1, keep_sl(pc), comm2_ref.at[pc])
            r.start()
            return r

        def s2_fin(pc, r):
            g = G[pc]
            r.wait()
            sl = (pl.ds(g["keep1"], HALF_ROW), cols(pc))
            acc_ref[sl] = acc_ref[sl] + comm2_ref[pc]

        def s3_start(pc):
            r = make_rdma(pc, 2, keep_sl(pc), keep_sl(pc))
            r.start()
            return r

        def writeback(pc, row0, k):
            sl = (pl.ds(row0, HALF_ROW), cols(pc))
            fbuf_ref[sl] = acc_ref[sl].astype(F32)
            cp = pltpu.make_async_copy(
                fbuf_ref.at[sl], out_ref.at[sl], wb_sems.at[k]
            )
            cp.start()
            return cp

        ew_cp.wait()
        ewb = ewv_ref[:, :, :].astype(BF)
        partialA = jnp.zeros((N_TOK, HALF_COL), F32)
        for j in range(EXP_PER_DEV):
            partialA = partialA + jnp.dot(
                xw[j], ewb[j, :, 0:HALF_COL], preferred_element_type=F32
            )
        acc_ref[:, 0:HALF_COL] = partialA.astype(BF)
        rd = [None] * N_PC
        rd[0] = s1_start(0)
        rd[1] = s1_start(1)

        partialB = jnp.zeros((N_TOK, HALF_COL), F32)
        for j in range(EXP_PER_DEV):
            partialB = partialB + jnp.dot(
                xw[j], ewb[j, :, HALF_COL:D_FF], preferred_element_type=F32
            )
        acc_ref[:, HALF_COL:D_FF] = partialB.astype(BF)
        rd[2] = s1_start(2)
        rd[3] = s1_start(3)

        wb = []
        for pc in (0, 2, 1, 3):
            s1_fin(pc, rd[pc])
            rd[pc] = s2_start(pc)
        for pc in (0, 2, 1, 3):
            s2_fin(pc, rd[pc])
            rd[pc] = s3_start(pc)
            wb.append(writeback(pc, G[pc]["keep1"], len(wb)))
        for pc in (0, 2, 1, 3):
            rd[pc].wait()
            wb.append(writeback(pc, G[pc]["send1"], len(wb)))
        for cp in wb:
            cp.wait()

    return pl.pallas_call(
        body,
        out_shape=jax.ShapeDtypeStruct((N_TOK, D_FF), F32),
        in_specs=[
            pl.BlockSpec(memory_space=pltpu.VMEM),
            pl.BlockSpec(memory_space=pltpu.VMEM),
            pl.BlockSpec(memory_space=pltpu.VMEM),
            pl.BlockSpec(memory_space=pltpu.MemorySpace.HBM),
        ],
        out_specs=pl.BlockSpec(memory_space=pltpu.MemorySpace.HBM),
        scratch_shapes=[
            pltpu.VMEM((N_TOK, D_FF), BF),
            pltpu.VMEM((N_TOK, D_FF), F32),
            pltpu.VMEM((EXP_PER_DEV, D_MODEL, D_FF), F32),
            pltpu.VMEM((N_PC, HALF_ROW, PW), BF),
            pltpu.VMEM((N_PC, HALF_ROW, PW), BF),
            pltpu.SemaphoreType.DMA((3 * N_PC,)),
            pltpu.SemaphoreType.DMA((3 * N_PC,)),
            pltpu.SemaphoreType.DMA,
            pltpu.SemaphoreType.DMA((2 * N_PC,)),
        ],
        compiler_params=pltpu.CompilerParams(collective_id=0),
    )(x, router_W, route_idx, expert_W)
